# baseline (speedup 1.0000x reference)
import jax
import jax.numpy as jnp
from jax.experimental import pallas as pl
from jax.experimental.pallas import tpu as pltpu

_VMEM_LIMIT = 48 * 1024 * 1024


def _norm_kernel(mean_ref, std_ref, x_ref, o_ref):
    rt = x_ref.shape[0]
    c = mean_ref.shape[0]
    ch = jax.lax.broadcasted_iota(jnp.int32, (rt, 1, 1), 0) % c
    scale = jnp.full((rt, 1, 1), 1.0 / (255.0 * std_ref[0]), jnp.float32)
    bias = jnp.full((rt, 1, 1), -mean_ref[0] / std_ref[0], jnp.float32)
    for k in range(1, c):
        scale = jnp.where(ch == k, 1.0 / (255.0 * std_ref[k]), scale)
        bias = jnp.where(ch == k, -mean_ref[k] / std_ref[k], bias)
    o_ref[...] = x_ref[...] * scale + bias


def kernel(img, mean, std):
    b, T, c, h, w = img.shape
    R = b * T * c
    x = img.reshape(R, h, w)

    row_tile = 24
    while R % row_tile != 0 or row_tile % c != 0:
        row_tile -= 1

    out = pl.pallas_call(
        _norm_kernel,
        out_shape=jax.ShapeDtypeStruct((R, h, w), jnp.float32),
        grid=(R // row_tile,),
        in_specs=[
            pl.BlockSpec(memory_space=pltpu.SMEM),
            pl.BlockSpec(memory_space=pltpu.SMEM),
            pl.BlockSpec((row_tile, h, w), lambda i: (i, 0, 0)),
        ],
        out_specs=pl.BlockSpec((row_tile, h, w), lambda i: (i, 0, 0)),
        compiler_params=pltpu.CompilerParams(
            dimension_semantics=("parallel",), vmem_limit_bytes=_VMEM_LIMIT),
    )(mean, std, x)
    return out.reshape(b, T, c, h, w)


# speedup vs baseline: 1.0967x; 1.0967x over previous
import jax
import jax.numpy as jnp
from jax.experimental import pallas as pl
from jax.experimental.pallas import tpu as pltpu

_VMEM_LIMIT = 48 * 1024 * 1024


def _norm_kernel(x_ref, scale_ref, bias_ref, o_ref):
    o_ref[...] = x_ref[...] * scale_ref[...] + bias_ref[...]


def kernel(img, mean, std):
    b, T, c, h, w = img.shape
    R = b * T * c
    x = img.reshape(R, h, w)

    scale_c = (1.0 / (255.0 * std)).astype(jnp.float32)
    bias_c = (-mean / std).astype(jnp.float32)
    scale = jnp.broadcast_to(scale_c[None, :], (b * T, c)).reshape(R, 1, 1)
    bias = jnp.broadcast_to(bias_c[None, :], (b * T, c)).reshape(R, 1, 1)

    row_tile = 32
    while R % row_tile != 0:
        row_tile -= 1

    out = pl.pallas_call(
        _norm_kernel,
        out_shape=jax.ShapeDtypeStruct((R, h, w), jnp.float32),
        grid=(R // row_tile,),
        in_specs=[
            pl.BlockSpec((row_tile, h, w), lambda i: (i, 0, 0)),
            pl.BlockSpec((row_tile, 1, 1), lambda i: (i, 0, 0)),
            pl.BlockSpec((row_tile, 1, 1), lambda i: (i, 0, 0)),
        ],
        out_specs=pl.BlockSpec((row_tile, h, w), lambda i: (i, 0, 0)),
        compiler_params=pltpu.CompilerParams(
            dimension_semantics=("parallel",), vmem_limit_bytes=_VMEM_LIMIT),
    )(x, scale, bias)
    return out.reshape(b, T, c, h, w)


# speedup vs baseline: 1.2046x; 1.0984x over previous
import jax
import jax.numpy as jnp
from jax.experimental import pallas as pl
from jax.experimental.pallas import tpu as pltpu

_VMEM_LIMIT = 60 * 1024 * 1024


def _norm_kernel(mean_ref, std_ref, x_ref, o_ref):
    rt = x_ref.shape[0]
    c = mean_ref.shape[0]
    ch = jax.lax.broadcasted_iota(jnp.int32, (rt, 1, 1), 0) % c
    scale = jnp.full((rt, 1, 1), 1.0 / (255.0 * std_ref[0]), jnp.float32)
    bias = jnp.full((rt, 1, 1), -mean_ref[0] / std_ref[0], jnp.float32)
    for k in range(1, c):
        scale = jnp.where(ch == k, 1.0 / (255.0 * std_ref[k]), scale)
        bias = jnp.where(ch == k, -mean_ref[k] / std_ref[k], bias)
    o_ref[...] = x_ref[...] * scale + bias


def kernel(img, mean, std):
    b, T, c, h, w = img.shape
    R = b * T * c
    x = img.reshape(R, h, w)

    row_tile = 48
    while R % row_tile != 0 or row_tile % c != 0:
        row_tile -= 1

    out = pl.pallas_call(
        _norm_kernel,
        out_shape=jax.ShapeDtypeStruct((R, h, w), jnp.float32),
        grid=(R // row_tile,),
        in_specs=[
            pl.BlockSpec(memory_space=pltpu.SMEM),
            pl.BlockSpec(memory_space=pltpu.SMEM),
            pl.BlockSpec((row_tile, h, w), lambda i: (i, 0, 0)),
        ],
        out_specs=pl.BlockSpec((row_tile, h, w), lambda i: (i, 0, 0)),
        compiler_params=pltpu.CompilerParams(
            dimension_semantics=("parallel",), vmem_limit_bytes=_VMEM_LIMIT),
    )(mean, std, x)
    return out.reshape(b, T, c, h, w)
